# baseline (speedup 1.0000x reference)
import functools

import jax
import jax.numpy as jnp
from jax import lax
from jax.experimental import pallas as pl
from jax.experimental.pallas import tpu as pltpu, tpu_sc as plsc

_CHUNK = 128


def _make_gather(B: int, D: int):
    info = plsc.get_sparse_core_info()
    NC, NS = info.num_cores, info.num_subcores
    NW = NC * NS
    assert B % (8 * NW) == 0
    b_per_w = B // NW
    n_chunks = b_per_w // _CHUNK
    assert n_chunks * _CHUNK == b_per_w
    mesh = plsc.VectorSubcoreMesh(core_axis_name="c", subcore_axis_name="s")

    @functools.partial(
        pl.kernel,
        mesh=mesh,
        out_type=jax.ShapeDtypeStruct((B, D), jnp.float32),
        compiler_params=pltpu.CompilerParams(
            needs_layout_passes=False,
            use_tc_tiling_on_sc=False,
        ),
        scratch_types=[
            pltpu.VMEM((b_per_w,), jnp.int32),
            pltpu.VMEM((b_per_w, D), jnp.float32),
            pltpu.SemaphoreType.DMA,
        ],
    )
    def gather_kernel(idx_hbm, table_hbm, out_hbm, idx_v, rows_v, sem):
        wid = lax.axis_index("s") * NC + lax.axis_index("c")
        base = wid * b_per_w
        pltpu.sync_copy(idx_hbm.at[pl.ds(base, b_per_w)], idx_v)
        copies = []
        for j in range(n_chunks):
            copies.append(
                pltpu.async_copy(
                    table_hbm.at[idx_v.at[pl.ds(j * _CHUNK, _CHUNK)]],
                    rows_v.at[pl.ds(j * _CHUNK, _CHUNK)],
                    sem,
                )
            )
        for c in copies:
            c.wait()
        pltpu.sync_copy(rows_v, out_hbm.at[pl.ds(base, b_per_w)])

    return gather_kernel


def kernel(batch_ids, latents):
    B = batch_ids.shape[0]
    D = latents.shape[1]
    idx = batch_ids.astype(jnp.int32)
    out = _make_gather(B, D)(idx, latents)
    return out.reshape(B, 1, 1, D)


# speedup vs baseline: 1.3745x; 1.3745x over previous
import functools

import jax
import jax.numpy as jnp
from jax.experimental import pallas as pl
from jax.experimental.pallas import tpu as pltpu

_R = 256


def _tc_gather(B: int, D: int):
    S = B // _R

    def body(idx_ref, table_ref, out_ref, sem):
        s = pl.program_id(0)
        copies = []
        for j in range(_R):
            row = idx_ref[s * _R + j]
            copies.append(
                pltpu.make_async_copy(
                    table_ref.at[pl.ds(row, 1), :],
                    out_ref.at[pl.ds(j, 1), :],
                    sem,
                )
            )
        for c in copies:
            c.start()
        for c in copies:
            c.wait()

    grid_spec = pltpu.PrefetchScalarGridSpec(
        num_scalar_prefetch=1,
        grid=(S,),
        in_specs=[pl.BlockSpec(memory_space=pl.ANY)],
        out_specs=pl.BlockSpec((_R, D), lambda s, idx_ref: (s, 0)),
        scratch_shapes=[pltpu.SemaphoreType.DMA],
    )
    return pl.pallas_call(
        body,
        grid_spec=grid_spec,
        out_shape=jax.ShapeDtypeStruct((B, D), jnp.float32),
    )


def kernel(batch_ids, latents):
    B = batch_ids.shape[0]
    D = latents.shape[1]
    idx = batch_ids.astype(jnp.int32)
    out = _tc_gather(B, D)(idx, latents)
    return out.reshape(B, 1, 1, D)


# speedup vs baseline: 1.5019x; 1.0927x over previous
import functools

import jax
import jax.numpy as jnp
from jax import lax
from jax.experimental import pallas as pl
from jax.experimental.pallas import tpu as pltpu, tpu_sc as plsc

_B_SC = 9216
_R_TC = 256


def _sc_gather(B: int, D: int):
    info = plsc.get_sparse_core_info()
    NC, NS = info.num_cores, info.num_subcores
    NW = NC * NS
    assert B % (8 * NW) == 0
    b_per_w = B // NW
    mesh = plsc.VectorSubcoreMesh(core_axis_name="c", subcore_axis_name="s")

    @functools.partial(
        pl.kernel,
        mesh=mesh,
        out_type=jax.ShapeDtypeStruct((B, D), jnp.float32),
        compiler_params=pltpu.CompilerParams(needs_layout_passes=False),
        scratch_types=[
            pltpu.VMEM((b_per_w,), jnp.int32),
            pltpu.VMEM((b_per_w, D), jnp.float32),
            pltpu.SemaphoreType.DMA,
        ],
    )
    def gather_kernel(idx_hbm, table_hbm, out_hbm, idx_v, rows_v, sem):
        wid = lax.axis_index("s") * NC + lax.axis_index("c")
        base = wid * b_per_w
        pltpu.sync_copy(idx_hbm.at[pl.ds(base, b_per_w)], idx_v)

        def fire(g):
            v = idx_v[pl.ds(g * 16, 16)]
            for l in range(16):
                pltpu.async_copy(table_hbm.at[v[l]], rows_v.at[g * 16 + l], sem)

        pl.loop(0, b_per_w // 16)(fire)
        pltpu.make_async_copy(out_hbm.at[pl.ds(0, b_per_w)], rows_v, sem).wait()
        pltpu.sync_copy(rows_v, out_hbm.at[pl.ds(base, b_per_w)])

    return gather_kernel


def _tc_gather(B: int, D: int):
    S = B // _R_TC
    assert S * _R_TC == B

    def body(idx_ref, table_ref, out_ref, sem):
        s = pl.program_id(0)
        copies = []
        for j in range(_R_TC):
            row = idx_ref[s * _R_TC + j]
            copies.append(
                pltpu.make_async_copy(
                    table_ref.at[pl.ds(row, 1), :],
                    out_ref.at[pl.ds(j, 1), :],
                    sem,
                )
            )
        for c in copies:
            c.start()
        for c in copies:
            c.wait()

    grid_spec = pltpu.PrefetchScalarGridSpec(
        num_scalar_prefetch=1,
        grid=(S,),
        in_specs=[pl.BlockSpec(memory_space=pl.ANY)],
        out_specs=pl.BlockSpec((_R_TC, D), lambda s, idx_ref: (s, 0)),
        scratch_shapes=[pltpu.SemaphoreType.DMA],
    )
    return pl.pallas_call(
        body,
        grid_spec=grid_spec,
        out_shape=jax.ShapeDtypeStruct((B, D), jnp.float32),
    )


def kernel(batch_ids, latents):
    B = batch_ids.shape[0]
    D = latents.shape[1]
    idx = batch_ids.astype(jnp.int32)
    sc_out = _sc_gather(_B_SC, D)(idx[:_B_SC], latents)
    tc_out = _tc_gather(B - _B_SC, D)(idx[_B_SC:], latents)
    out = jnp.concatenate([sc_out, tc_out], axis=0)
    return out.reshape(B, 1, 1, D)


# speedup vs baseline: 1.7073x; 1.1368x over previous
import functools

import jax
import jax.numpy as jnp
from jax import lax
from jax.experimental import pallas as pl
from jax.experimental.pallas import tpu as pltpu, tpu_sc as plsc

_B_SC = 9216
_R_TC = 256


def _sc_gather(B: int, D: int):
    info = plsc.get_sparse_core_info()
    NC, NS = info.num_cores, info.num_subcores
    NW = NC * NS
    assert B % (8 * NW) == 0
    b_per_w = B // NW
    mesh = plsc.VectorSubcoreMesh(core_axis_name="c", subcore_axis_name="s")

    @functools.partial(
        pl.kernel,
        mesh=mesh,
        out_type=jax.ShapeDtypeStruct((B, D), jnp.float32),
        compiler_params=pltpu.CompilerParams(needs_layout_passes=False),
        scratch_types=[
            pltpu.VMEM((b_per_w,), jnp.int32),
            pltpu.VMEM((b_per_w, D), jnp.float32),
            pltpu.SemaphoreType.DMA,
        ],
    )
    def gather_kernel(idx_hbm, table_hbm, out_hbm, idx_v, rows_v, sem):
        wid = lax.axis_index("s") * NC + lax.axis_index("c")
        base = wid * b_per_w
        pltpu.sync_copy(idx_hbm.at[pl.ds(base, b_per_w)], idx_v)

        def fire(g):
            v = idx_v[pl.ds(g * 16, 16)]
            for l in range(16):
                pltpu.async_copy(table_hbm.at[v[l]], rows_v.at[g * 16 + l], sem)

        pl.loop(0, b_per_w // 16)(fire)
        pltpu.make_async_copy(out_hbm.at[pl.ds(0, b_per_w)], rows_v, sem).wait()
        pltpu.sync_copy(rows_v, out_hbm.at[pl.ds(base, b_per_w)])

    return gather_kernel


def _tc_gather(B: int, D: int):
    S = B // _R_TC
    assert S * _R_TC == B

    def body(idx_ref, table_ref, out_ref, sem):
        s = pl.program_id(0)
        copies = []
        for j in range(_R_TC):
            row = idx_ref[s * _R_TC + j]
            copies.append(
                pltpu.make_async_copy(
                    table_ref.at[pl.ds(row, 1), :],
                    out_ref.at[pl.ds(j, 1), :],
                    sem,
                )
            )
        for c in copies:
            c.start()
        for c in copies:
            c.wait()

    grid_spec = pltpu.PrefetchScalarGridSpec(
        num_scalar_prefetch=1,
        grid=(S,),
        in_specs=[pl.BlockSpec(memory_space=pl.ANY)],
        out_specs=pl.BlockSpec((_R_TC, D), lambda s, idx_ref: (s, 0)),
        scratch_shapes=[pltpu.SemaphoreType.DMA],
    )
    return pl.pallas_call(
        body,
        grid_spec=grid_spec,
        out_shape=jax.ShapeDtypeStruct((B, D), jnp.float32),
    )


def kernel(batch_ids, latents):
    B = batch_ids.shape[0]
    D = latents.shape[1]
    idx = batch_ids.astype(jnp.int32)
    out = _sc_gather(B, D)(idx, latents)
    return out.reshape(B, 1, 1, D)


# speedup vs baseline: 5.8267x; 3.4128x over previous
import functools

import jax
import jax.numpy as jnp
from jax import lax
from jax.experimental import pallas as pl
from jax.experimental.pallas import tpu as pltpu, tpu_sc as plsc

_C = 64


def _sc_gather_t(B: int, D: int, N: int):
    info = plsc.get_sparse_core_info()
    NC, NS = info.num_cores, info.num_subcores
    NW = NC * NS
    assert B % (8 * NW) == 0 and D == 64
    b_per_w = B // NW
    n_chunks = b_per_w // _C
    assert n_chunks * _C == b_per_w
    mesh = plsc.VectorSubcoreMesh(core_axis_name="c", subcore_axis_name="s")

    @functools.partial(
        pl.kernel,
        mesh=mesh,
        out_type=jax.ShapeDtypeStruct((8, 8, B), jnp.float32),
        compiler_params=pltpu.CompilerParams(needs_layout_passes=False),
        scratch_types=[
            pltpu.VMEM((b_per_w,), jnp.int32),
            pltpu.VMEM((8, 8, _C * 16), jnp.float32),
            pltpu.VMEM((8, 8, b_per_w), jnp.float32),
            pltpu.SemaphoreType.DMA,
        ],
    )
    def gather_kernel(idx_hbm, table_hbm, out_hbm, idx_v, st_v, ob_v, sem):
        wid = lax.axis_index("s") * NC + lax.axis_index("c")
        base = wid * b_per_w
        pltpu.sync_copy(idx_hbm.at[pl.ds(base, b_per_w)], idx_v)

        lane = lax.iota(jnp.int32, 16)
        tj0, s0 = lane >> 3, lane & 7
        tj1, s1 = (lane + 16) >> 3, lane & 7
        tj2, s2 = (lane + 32) >> 3, lane & 7
        tj3, s3 = (lane + 48) >> 3, lane & 7
        groups = ((tj0, s0), (tj1, s1), (tj2, s2), (tj3, s3))

        def chunk_body(k):
            def fire(g):
                v = idx_v[pl.ds(k * _C + g * 16, 16)]
                vb = jax.lax.shift_right_logical(v, 4)
                for l in range(16):
                    pltpu.async_copy(
                        table_hbm.at[:, :, pl.ds(vb[l] * 16, 16)],
                        st_v.at[:, :, pl.ds((g * 16 + l) * 16, 16)],
                        sem,
                    )

            pl.loop(0, _C // 16)(fire)
            pltpu.make_async_copy(
                table_hbm.at[:, :, pl.ds(0, _C * 16)], st_v, sem
            ).wait()

            def extract(g):
                v = idx_v[pl.ds(k * _C + g * 16, 16)]
                vc = jax.lax.bitwise_and(v, 15)
                for l in range(16):
                    j = g * 16 + l
                    src_lane = jnp.full((16,), j * 16, jnp.int32) + vc[l]
                    dst_lane = jnp.full((16,), k * _C + j, jnp.int32)
                    for tjv, sv in groups:
                        val = plsc.load_gather(st_v, [tjv, sv, src_lane])
                        plsc.store_scatter(ob_v, [tjv, sv, dst_lane], val)

            pl.loop(0, _C // 16)(extract)

        pl.loop(0, n_chunks)(chunk_body)
        pltpu.sync_copy(ob_v, out_hbm.at[:, :, pl.ds(base, b_per_w)])

    return gather_kernel


def kernel(batch_ids, latents):
    B = batch_ids.shape[0]
    N, D = latents.shape
    idx = batch_ids.astype(jnp.int32)
    table_t = latents.T.reshape(8, 8, N)
    out_t = _sc_gather_t(B, D, N)(idx, table_t)
    return out_t.reshape(D, B).T.reshape(B, 1, 1, D)
